# Initial kernel scaffold; baseline (speedup 1.0000x reference)
import jax
import jax.numpy as jnp
from jax.experimental import pallas as pl


def kernel(queries, keys, W1, b1, W2, b2, W3, b3, W4, b4):
    raise NotImplementedError("write your pallas kernel here")


# speedup vs baseline: 2.5102x; 2.5102x over previous
import jax
import jax.numpy as jnp
from jax.experimental import pallas as pl
from jax.experimental.pallas import tpu as pltpu

_TK = 2048


def _relu(x):
    return jnp.maximum(x, 0.0)


def _dot(a, b):
    return jnp.dot(a.astype(jnp.bfloat16), b.astype(jnp.bfloat16),
                   preferred_element_type=jnp.float32)


def _bf(x):
    return x.astype(jnp.bfloat16).astype(jnp.float32)


def _body(qT_ref, keys_ref,
          W1_ref, b1_ref, W2_ref, b2_ref, W3_ref, b3_ref, W4_ref, b4_ref,
          W1T_ref, b1c_ref, W2T_ref, b2c_ref, W3T_ref, b3c_ref, W4T_ref, b4c_ref,
          prob_ref, idx_ref, mind_ref,
          qx0_s, qx1_s, q2_s, m_s, s_s, i_s):
    step = pl.program_id(0)
    nsteps = pl.num_programs(0)
    tk = keys_ref.shape[0]

    @pl.when(step == 0)
    def _init():
        qT = qT_ref[...]
        a1 = _relu(_dot(W1T_ref[...], qT) + b1c_ref[...])
        a2 = _relu(_dot(W2T_ref[...], a1) + b2c_ref[...])
        a3 = _relu(_dot(W3T_ref[...], a2) + b3c_ref[...])
        a4 = _dot(W4T_ref[...], a3) + b4c_ref[...]
        qx0_s[...] = a4[0:1, :]
        qx1_s[...] = a4[1:2, :]
        q2_s[...] = a4[0:1, :] * a4[0:1, :] + a4[1:2, :] * a4[1:2, :]
        m_s[...] = jnp.full(m_s.shape, jnp.inf, jnp.float32)
        s_s[...] = jnp.zeros(s_s.shape, jnp.float32)
        i_s[...] = jnp.zeros(i_s.shape, jnp.int32)

    k_in = keys_ref[...]
    a1 = _relu(_dot(k_in, W1_ref[...]) + b1_ref[...])
    a2 = _relu(_dot(a1, W2_ref[...]) + b2_ref[...])
    a3 = _relu(_dot(a2, W3_ref[...]) + b3_ref[...])
    kx = _dot(a3, W4_ref[...]) + b4_ref[...]

    kx0 = kx[:, 0:1]
    kx1 = kx[:, 1:2]
    k2 = kx0 * kx0 + kx1 * kx1
    p = _bf(kx0) * _bf(qx0_s[...]) + _bf(kx1) * _bf(qx1_s[...])
    d2 = (q2_s[...] - 2.0 * p) + k2
    dist = jnp.sqrt(jnp.maximum(d2, 0.0) + 1e-12)

    tm = jnp.min(dist, axis=0, keepdims=True)
    m_old = m_s[...]
    m_new = jnp.minimum(m_old, tm)
    e = jnp.exp(m_new - dist)
    ts = jnp.sum(e, axis=0, keepdims=True)
    s_s[...] = s_s[...] * jnp.exp(m_new - m_old) + ts
    m_s[...] = m_new

    ii = jax.lax.broadcasted_iota(jnp.int32, dist.shape, 0) + step * tk
    lidx = jnp.min(jnp.where(dist == tm, ii, jnp.int32(2147483647)),
                   axis=0, keepdims=True)
    i_s[...] = jnp.where(tm < m_old, lidx, i_s[...])

    @pl.when(step == nsteps - 1)
    def _fin():
        mind_ref[...] = m_s[...]
        idx_ref[...] = i_s[...]
        prob_ref[...] = -jnp.log(s_s[...])


def kernel(queries, keys, W1, b1, W2, b2, W3, b3, W4, b4):
    Q = queries.shape[0]
    K = keys.shape[0]
    assert K % _TK == 0

    qT = queries.T
    b1r, b2r, b3r, b4r = (b.reshape(1, -1) for b in (b1, b2, b3, b4))
    W1T, W2T, W3T, W4T = W1.T, W2.T, W3.T, W4.T
    b1c, b2c, b3c, b4c = (b.reshape(-1, 1) for b in (b1, b2, b3, b4))

    def _full(a):
        return pl.BlockSpec(a.shape, lambda i: (0,) * a.ndim)

    def row(dt):
        return pl.BlockSpec((1, Q), lambda i: (0, 0))

    prob, idx, mind = pl.pallas_call(
        _body,
        grid=(K // _TK,),
        in_specs=[
            _full(qT),
            pl.BlockSpec((_TK, 2), lambda i: (i, 0)),
            _full(W1), _full(b1r), _full(W2), _full(b2r),
            _full(W3), _full(b3r), _full(W4), _full(b4r),
            _full(W1T), _full(b1c), _full(W2T), _full(b2c),
            _full(W3T), _full(b3c), _full(W4T), _full(b4c),
        ],
        out_specs=[row(jnp.float32), row(jnp.int32), row(jnp.float32)],
        out_shape=[
            jax.ShapeDtypeStruct((1, Q), jnp.float32),
            jax.ShapeDtypeStruct((1, Q), jnp.int32),
            jax.ShapeDtypeStruct((1, Q), jnp.float32),
        ],
        scratch_shapes=[
            pltpu.VMEM((1, Q), jnp.float32),
            pltpu.VMEM((1, Q), jnp.float32),
            pltpu.VMEM((1, Q), jnp.float32),
            pltpu.VMEM((1, Q), jnp.float32),
            pltpu.VMEM((1, Q), jnp.float32),
            pltpu.VMEM((1, Q), jnp.int32),
        ],
    )(qT, keys, W1, b1r, W2, b2r, W3, b3r, W4, b4r,
      W1T, b1c, W2T, b2c, W3T, b3c, W4T, b4c)

    return prob.reshape(Q), idx.reshape(Q), mind.reshape(Q)


# speedup vs baseline: 3.1943x; 1.2725x over previous
import jax
import jax.numpy as jnp
from jax.experimental import pallas as pl
from jax.experimental.pallas import tpu as pltpu

_TK = 2048


def _relu(x):
    return jnp.maximum(x, 0.0)


def _dot(a, b):
    return jnp.dot(a.astype(jnp.bfloat16), b.astype(jnp.bfloat16),
                   preferred_element_type=jnp.float32)


def _body(qT_ref, keys_ref,
          W1_ref, b1_ref, W2_ref, b2_ref, W3_ref, b3_ref, W4_ref, b4_ref,
          W1T_ref, b1c_ref, W2T_ref, b2c_ref, W3T_ref, b3c_ref, W4T_ref, b4c_ref,
          prob_ref, idx_ref, mind_ref,
          qxtb_s, q2_s, m_s, m2_s, s_s, i_s):
    step = pl.program_id(0)
    nsteps = pl.num_programs(0)
    tk = keys_ref.shape[0]

    @pl.when(step == 0)
    def _init():
        qT = qT_ref[...]
        a1 = _relu(_dot(W1T_ref[...], qT) + b1c_ref[...])
        a2 = _relu(_dot(W2T_ref[...], a1) + b2c_ref[...])
        a3 = _relu(_dot(W3T_ref[...], a2) + b3c_ref[...])
        a4 = _dot(W4T_ref[...], a3) + b4c_ref[...]
        qxtb_s[...] = a4.astype(jnp.bfloat16)
        q2_s[...] = a4[0:1, :] * a4[0:1, :] + a4[1:2, :] * a4[1:2, :]
        m_s[...] = jnp.full(m_s.shape, jnp.inf, jnp.float32)
        m2_s[...] = jnp.full(m2_s.shape, jnp.inf, jnp.float32)
        s_s[...] = jnp.zeros(s_s.shape, jnp.float32)
        i_s[...] = jnp.zeros(i_s.shape, jnp.int32)

    k_in = keys_ref[...]
    a1 = _relu(_dot(k_in, W1_ref[...]) + b1_ref[...])
    a2 = _relu(_dot(a1, W2_ref[...]) + b2_ref[...])
    a3 = _relu(_dot(a2, W3_ref[...]) + b3_ref[...])
    kx = _dot(a3, W4_ref[...]) + b4_ref[...]

    kx0 = kx[:, 0:1]
    kx1 = kx[:, 1:2]
    k2 = kx0 * kx0 + kx1 * kx1
    p = jnp.dot(kx.astype(jnp.bfloat16), qxtb_s[...],
                preferred_element_type=jnp.float32)
    d2 = (q2_s[...] - 2.0 * p) + k2
    d2p = jnp.maximum(d2, 0.0) + 1e-12

    tm2 = jnp.min(d2p, axis=0, keepdims=True)
    m2_old = m2_s[...]
    m2_new = jnp.minimum(m2_old, tm2)
    m_old = m_s[...]
    m_new = jnp.sqrt(m2_new)
    dist = d2p * jax.lax.rsqrt(d2p)
    e = jnp.exp(m_new - dist)
    ts = jnp.sum(e, axis=0, keepdims=True)
    s_s[...] = s_s[...] * jnp.exp(m_new - m_old) + ts
    m_s[...] = m_new
    m2_s[...] = m2_new

    ii = jax.lax.broadcasted_iota(jnp.int32, d2p.shape, 0) + step * tk
    lidx = jnp.min(jnp.where(d2p == tm2, ii, jnp.int32(2147483647)),
                   axis=0, keepdims=True)
    i_s[...] = jnp.where(tm2 < m2_old, lidx, i_s[...])

    @pl.when(step == nsteps - 1)
    def _fin():
        mind_ref[...] = m_s[...]
        idx_ref[...] = i_s[...]
        prob_ref[...] = -jnp.log(s_s[...])


def kernel(queries, keys, W1, b1, W2, b2, W3, b3, W4, b4):
    Q = queries.shape[0]
    K = keys.shape[0]
    assert K % _TK == 0

    qT = queries.T
    b1r, b2r, b3r, b4r = (b.reshape(1, -1) for b in (b1, b2, b3, b4))
    W1T, W2T, W3T, W4T = W1.T, W2.T, W3.T, W4.T
    b1c, b2c, b3c, b4c = (b.reshape(-1, 1) for b in (b1, b2, b3, b4))

    def _full(a):
        return pl.BlockSpec(a.shape, lambda i: (0,) * a.ndim)

    def row(dt):
        return pl.BlockSpec((1, Q), lambda i: (0, 0))

    prob, idx, mind = pl.pallas_call(
        _body,
        grid=(K // _TK,),
        in_specs=[
            _full(qT),
            pl.BlockSpec((_TK, 2), lambda i: (i, 0)),
            _full(W1), _full(b1r), _full(W2), _full(b2r),
            _full(W3), _full(b3r), _full(W4), _full(b4r),
            _full(W1T), _full(b1c), _full(W2T), _full(b2c),
            _full(W3T), _full(b3c), _full(W4T), _full(b4c),
        ],
        out_specs=[row(jnp.float32), row(jnp.int32), row(jnp.float32)],
        out_shape=[
            jax.ShapeDtypeStruct((1, Q), jnp.float32),
            jax.ShapeDtypeStruct((1, Q), jnp.int32),
            jax.ShapeDtypeStruct((1, Q), jnp.float32),
        ],
        scratch_shapes=[
            pltpu.VMEM((2, Q), jnp.bfloat16),
            pltpu.VMEM((1, Q), jnp.float32),
            pltpu.VMEM((1, Q), jnp.float32),
            pltpu.VMEM((1, Q), jnp.float32),
            pltpu.VMEM((1, Q), jnp.float32),
            pltpu.VMEM((1, Q), jnp.int32),
        ],
    )(qT, keys, W1, b1r, W2, b2r, W3, b3r, W4, b4r,
      W1T, b1c, W2T, b2c, W3T, b3c, W4T, b4c)

    return prob.reshape(Q), idx.reshape(Q), mind.reshape(Q)


# speedup vs baseline: 3.4499x; 1.0800x over previous
import jax
import jax.numpy as jnp
from jax.experimental import pallas as pl
from jax.experimental.pallas import tpu as pltpu

_TK = 4096


def _relu(x):
    return jnp.maximum(x, 0.0)


def _dot(a, b):
    return jnp.dot(a.astype(jnp.bfloat16), b.astype(jnp.bfloat16),
                   preferred_element_type=jnp.float32)


def _body(qT_ref, keys_ref, iota_ref,
          W1_ref, b1_ref, W2_ref, b2_ref, W3_ref, b3_ref, W4_ref, b4_ref,
          W1T_ref, b1c_ref, W2T_ref, b2c_ref, W3T_ref, b3c_ref, W4T_ref, b4c_ref,
          prob_ref, idx_ref, mind_ref,
          qxtb_s, q2_s, m_s, m2_s, s_s, i_s):
    step = pl.program_id(0)
    nsteps = pl.num_programs(0)
    tk = keys_ref.shape[0]

    @pl.when(step == 0)
    def _init():
        qT = qT_ref[...]
        a1 = _relu(_dot(W1T_ref[...], qT) + b1c_ref[...])
        a2 = _relu(_dot(W2T_ref[...], a1) + b2c_ref[...])
        a3 = _relu(_dot(W3T_ref[...], a2) + b3c_ref[...])
        a4 = _dot(W4T_ref[...], a3) + b4c_ref[...]
        qxtb_s[...] = a4.astype(jnp.bfloat16)
        q2_s[...] = a4[0:1, :] * a4[0:1, :] + a4[1:2, :] * a4[1:2, :]
        m_s[...] = jnp.full(m_s.shape, jnp.inf, jnp.float32)
        m2_s[...] = jnp.full(m2_s.shape, jnp.inf, jnp.float32)
        s_s[...] = jnp.zeros(s_s.shape, jnp.float32)
        i_s[...] = jnp.zeros(i_s.shape, jnp.int32)

    k_in = keys_ref[...]
    a1 = _relu(_dot(k_in, W1_ref[...]) + b1_ref[...])
    a2 = _relu(_dot(a1, W2_ref[...]) + b2_ref[...])
    a3 = _relu(_dot(a2, W3_ref[...]) + b3_ref[...])
    kx = _dot(a3, W4_ref[...]) + b4_ref[...]

    kx0 = kx[:, 0:1]
    kx1 = kx[:, 1:2]
    k2 = kx0 * kx0 + kx1 * kx1
    p = jnp.dot(kx.astype(jnp.bfloat16), qxtb_s[...],
                preferred_element_type=jnp.float32)
    d2 = (q2_s[...] - 2.0 * p) + k2
    d2p = jnp.maximum(d2, 0.0) + 1e-12

    tm2 = jnp.min(d2p, axis=0, keepdims=True)
    m2_old = m2_s[...]
    m2_new = jnp.minimum(m2_old, tm2)
    m_old = m_s[...]
    m_new = jnp.sqrt(m2_new)
    dist = d2p * jax.lax.rsqrt(d2p)
    e = jnp.exp(m_new - dist)
    ts = jnp.sum(e, axis=0, keepdims=True)
    s_s[...] = s_s[...] * jnp.exp(m_new - m_old) + ts
    m_s[...] = m_new
    m2_s[...] = m2_new

    lidxf = jnp.min(jnp.where(d2p == tm2, iota_ref[...], jnp.float32(3.4e38)),
                    axis=0, keepdims=True)
    lidx = lidxf.astype(jnp.int32) + step * tk
    i_s[...] = jnp.where(tm2 < m2_old, lidx, i_s[...])

    @pl.when(step == nsteps - 1)
    def _fin():
        mind_ref[...] = m_s[...]
        idx_ref[...] = i_s[...]
        prob_ref[...] = -jnp.log(s_s[...])


def kernel(queries, keys, W1, b1, W2, b2, W3, b3, W4, b4):
    Q = queries.shape[0]
    K = keys.shape[0]
    assert K % _TK == 0

    qT = queries.T
    iotaf = jnp.arange(_TK, dtype=jnp.float32).reshape(_TK, 1)
    b1r, b2r, b3r, b4r = (b.reshape(1, -1) for b in (b1, b2, b3, b4))
    W1T, W2T, W3T, W4T = W1.T, W2.T, W3.T, W4.T
    b1c, b2c, b3c, b4c = (b.reshape(-1, 1) for b in (b1, b2, b3, b4))

    def _full(a):
        return pl.BlockSpec(a.shape, lambda i: (0,) * a.ndim)

    def row(dt):
        return pl.BlockSpec((1, Q), lambda i: (0, 0))

    prob, idx, mind = pl.pallas_call(
        _body,
        grid=(K // _TK,),
        in_specs=[
            _full(qT),
            pl.BlockSpec((_TK, 2), lambda i: (i, 0)),
            _full(iotaf),
            _full(W1), _full(b1r), _full(W2), _full(b2r),
            _full(W3), _full(b3r), _full(W4), _full(b4r),
            _full(W1T), _full(b1c), _full(W2T), _full(b2c),
            _full(W3T), _full(b3c), _full(W4T), _full(b4c),
        ],
        out_specs=[row(jnp.float32), row(jnp.int32), row(jnp.float32)],
        out_shape=[
            jax.ShapeDtypeStruct((1, Q), jnp.float32),
            jax.ShapeDtypeStruct((1, Q), jnp.int32),
            jax.ShapeDtypeStruct((1, Q), jnp.float32),
        ],
        scratch_shapes=[
            pltpu.VMEM((2, Q), jnp.bfloat16),
            pltpu.VMEM((1, Q), jnp.float32),
            pltpu.VMEM((1, Q), jnp.float32),
            pltpu.VMEM((1, Q), jnp.float32),
            pltpu.VMEM((1, Q), jnp.float32),
            pltpu.VMEM((1, Q), jnp.int32),
        ],
    )(qT, keys, iotaf, W1, b1r, W2, b2r, W3, b3r, W4, b4r,
      W1T, b1c, W2T, b2c, W3T, b3c, W4T, b4c)

    return prob.reshape(Q), idx.reshape(Q), mind.reshape(Q)


# speedup vs baseline: 3.6511x; 1.0583x over previous
import jax
import jax.numpy as jnp
from jax.experimental import pallas as pl
from jax.experimental.pallas import tpu as pltpu

_TK = 4096


def _relu(x):
    return jnp.maximum(x, 0.0)


def _dot(a, b):
    return jnp.dot(a.astype(jnp.bfloat16), b.astype(jnp.bfloat16),
                   preferred_element_type=jnp.float32)


def _body(qT_ref, keys_ref,
          W1_ref, b1_ref, W2_ref, b2_ref, W3_ref, b3_ref, W4_ref, b4_ref,
          W1T_ref, b1c_ref, W2T_ref, b2c_ref, W3T_ref, b3c_ref, W4T_ref, b4c_ref,
          prob_ref, idx_ref, mind_ref,
          qxtb_s, q2_s, m_s, m2_s, s_s, i_s):
    step = pl.program_id(0)
    nsteps = pl.num_programs(0)
    tk = keys_ref.shape[0]

    @pl.when(step == 0)
    def _init():
        qT = qT_ref[...]
        a1 = _relu(_dot(W1T_ref[...], qT) + b1c_ref[...])
        a2 = _relu(_dot(W2T_ref[...], a1) + b2c_ref[...])
        a3 = _relu(_dot(W3T_ref[...], a2) + b3c_ref[...])
        a4 = _dot(W4T_ref[...], a3) + b4c_ref[...]
        qxtb_s[...] = a4.astype(jnp.bfloat16)
        q2_s[...] = a4[0:1, :] * a4[0:1, :] + a4[1:2, :] * a4[1:2, :]
        m_s[...] = jnp.full(m_s.shape, jnp.inf, jnp.float32)
        m2_s[...] = jnp.full(m2_s.shape, jnp.inf, jnp.float32)
        s_s[...] = jnp.zeros(s_s.shape, jnp.float32)
        i_s[...] = jnp.zeros(i_s.shape, jnp.int32)

    k_in = keys_ref[...]
    a1 = _relu(_dot(k_in, W1_ref[...]) + b1_ref[...])
    a2 = _relu(_dot(a1, W2_ref[...]) + b2_ref[...])
    a3 = _relu(_dot(a2, W3_ref[...]) + b3_ref[...])
    kx = _dot(a3, W4_ref[...]) + b4_ref[...]

    kx0 = kx[:, 0:1]
    kx1 = kx[:, 1:2]
    k2 = kx0 * kx0 + kx1 * kx1
    p = jnp.dot(kx.astype(jnp.bfloat16), qxtb_s[...],
                preferred_element_type=jnp.float32)
    d2 = (q2_s[...] - 2.0 * p) + k2
    d2p = jnp.maximum(d2, 0.0) + 1e-12

    tm2 = jnp.min(d2p, axis=0, keepdims=True)
    m2_old = m2_s[...]
    m2_new = jnp.minimum(m2_old, tm2)
    m_old = m_s[...]
    m_new = jnp.sqrt(m2_new)
    dist = d2p * jax.lax.rsqrt(d2p)
    e = jnp.exp(m_new - dist)
    ts = jnp.sum(e, axis=0, keepdims=True)
    s_s[...] = s_s[...] * jnp.exp(m_new - m_old) + ts
    m_s[...] = m_new
    m2_s[...] = m2_new

    lidx = jnp.argmin(d2p, axis=0).astype(jnp.int32).reshape(1, -1) + step * tk
    i_s[...] = jnp.where(tm2 < m2_old, lidx, i_s[...])

    @pl.when(step == nsteps - 1)
    def _fin():
        mind_ref[...] = m_s[...]
        idx_ref[...] = i_s[...]
        prob_ref[...] = -jnp.log(s_s[...])


def kernel(queries, keys, W1, b1, W2, b2, W3, b3, W4, b4):
    Q = queries.shape[0]
    K = keys.shape[0]
    assert K % _TK == 0

    qT = queries.T
    b1r, b2r, b3r, b4r = (b.reshape(1, -1) for b in (b1, b2, b3, b4))
    W1T, W2T, W3T, W4T = W1.T, W2.T, W3.T, W4.T
    b1c, b2c, b3c, b4c = (b.reshape(-1, 1) for b in (b1, b2, b3, b4))

    def _full(a):
        return pl.BlockSpec(a.shape, lambda i: (0,) * a.ndim)

    def row(dt):
        return pl.BlockSpec((1, Q), lambda i: (0, 0))

    prob, idx, mind = pl.pallas_call(
        _body,
        grid=(K // _TK,),
        in_specs=[
            _full(qT),
            pl.BlockSpec((_TK, 2), lambda i: (i, 0)),
            _full(W1), _full(b1r), _full(W2), _full(b2r),
            _full(W3), _full(b3r), _full(W4), _full(b4r),
            _full(W1T), _full(b1c), _full(W2T), _full(b2c),
            _full(W3T), _full(b3c), _full(W4T), _full(b4c),
        ],
        out_specs=[row(jnp.float32), row(jnp.int32), row(jnp.float32)],
        out_shape=[
            jax.ShapeDtypeStruct((1, Q), jnp.float32),
            jax.ShapeDtypeStruct((1, Q), jnp.int32),
            jax.ShapeDtypeStruct((1, Q), jnp.float32),
        ],
        scratch_shapes=[
            pltpu.VMEM((2, Q), jnp.bfloat16),
            pltpu.VMEM((1, Q), jnp.float32),
            pltpu.VMEM((1, Q), jnp.float32),
            pltpu.VMEM((1, Q), jnp.float32),
            pltpu.VMEM((1, Q), jnp.float32),
            pltpu.VMEM((1, Q), jnp.int32),
        ],
    )(qT, keys, W1, b1r, W2, b2r, W3, b3r, W4, b4r,
      W1T, b1c, W2T, b2c, W3T, b3c, W4T, b4c)

    return prob.reshape(Q), idx.reshape(Q), mind.reshape(Q)
